# baseline (speedup 1.0000x reference)
import functools

import jax
import jax.numpy as jnp
import numpy as np
from jax import lax
from jax.experimental import pallas as pl
from jax.experimental.pallas import tpu as pltpu
from jax.experimental.pallas import tpu_sc as plsc

N = 10000
E = 640000
MAX_AT = 119
CUTOFF = 5.0
NB = 16
NS = 64
NV = 16

NC = 2
NSUB = 16
NW = NC * NSUB
EW = E // NW
C = 80
NCHUNK = EW // C
GP = C // 16
N_PAD = 10240
NSTRIPE = N_PAD // NSUB

_SQ25 = float(np.sqrt(2.0 / CUTOFF))
_SQRT3 = float(np.sqrt(3.0))
_PI = float(np.pi)

_mesh = plsc.VectorSubcoreMesh(core_axis_name="c", subcore_axis_name="s")


def _sc_body(xs_h, ys_h, zs_h, src_h, dst_h,
             rbf_h, rsh_h, acc_h,
             xs, ys, zs,
             sbuf0, sbuf1, dbuf0, dbuf1, pay0, pay1,
             rbfb0, rbfb1, rshb0, rshb1, accs,
             semi0, semi1, semw0, semw1):
    cid = lax.axis_index("c")
    sid = lax.axis_index("s")
    wid = sid * NC + cid
    base = wid * EW
    sbuf = (sbuf0, sbuf1)
    dbuf = (dbuf0, dbuf1)
    pay = (pay0, pay1)
    rbfb = (rbfb0, rbfb1)
    rshb = (rshb0, rshb1)
    semi = (semi0, semi1)
    semw = (semw0, semw1)

    pltpu.sync_copy(xs_h, xs)
    pltpu.sync_copy(ys_h, ys)
    pltpu.sync_copy(zs_h, zs)
    stripe = sid * NSTRIPE
    zv = jnp.full((16,), 0.0, jnp.float32)
    for i in range(C):
        for j in range(NS // 16):
            pay0[i, pl.ds(j * 16, 16)] = zv
    for j in range(NSTRIPE // C):
        pltpu.sync_copy(pay0, accs.at[pl.ds(stripe + j * C, C)])
    plsc.subcore_barrier()

    lanes = lax.iota(jnp.int32, 16)

    for b in (0, 1):
        pltpu.async_copy(src_h.at[pl.ds(base + b * C, C)], sbuf[b], semi[b])
        pltpu.async_copy(dst_h.at[pl.ds(base + b * C, C)], dbuf[b], semi[b])

    def do_chunk(t, b):
        eb = base + t * C
        pltpu.make_async_copy(src_h.at[pl.ds(eb, C)], sbuf[b], semi[b]).wait()
        pltpu.make_async_copy(dst_h.at[pl.ds(eb, C)], dbuf[b], semi[b]).wait()
        @pl.when(t >= 2)
        def _drain():
            eo = eb - 2 * C
            pltpu.make_async_copy(rbfb[b], rbf_h.at[:, pl.ds(eo, C)],
                                  semw[b]).wait()
            pltpu.make_async_copy(rshb[b], rsh_h.at[:, pl.ds(eo, C)],
                                  semw[b]).wait()
        for g in range(GP):
            si = sbuf[b][pl.ds(g * 16, 16)]
            di = dbuf[b][pl.ds(g * 16, 16)]
            vx = plsc.load_gather(xs, [di]) - plsc.load_gather(xs, [si])
            vy = plsc.load_gather(ys, [di]) - plsc.load_gather(ys, [si])
            vz = plsc.load_gather(zs, [di]) - plsc.load_gather(zs, [si])
            d2 = vx * vx + vy * vy + vz * vz + 1e-12
            y = plsc.bitcast(0x5F3759DF - jnp.right_shift(plsc.bitcast(d2, jnp.int32), 1),
                             jnp.float32)
            y = y * (1.5 - 0.5 * d2 * y * y)
            y = y * (1.5 - 0.5 * d2 * y * y)
            rinv = y * (1.5 - 0.5 * d2 * y * y)
            dist = d2 * rinv
            u = dist * (1.0 / CUTOFF)
            inside = u < 1.0
            u2 = u * u
            u5 = u2 * u2 * u
            fcut = jnp.where(inside, 1.0 + u5 * (-21.0 + u * (35.0 - 15.0 * u)), 0.0)
            uc = jnp.minimum(u, 1.0)
            flip = uc > 0.5
            w = jnp.where(flip, 1.0 - uc, uc)
            r = _PI * w
            r2 = r * r
            s1 = r * (1.0 + r2 * (-1.6666654611e-1 + r2 * (8.3321608736e-3
                                                           + r2 * -1.9515295891e-4)))
            c1m = 1.0 + r2 * (-0.4999999724 + r2 * (4.166654929e-2
                                                    + r2 * (-1.388731625e-3
                                                            + r2 * 2.443315711e-5)))
            c1 = jnp.where(flip, 0.0 - c1m, c1m)
            gsc = _SQ25 * rinv * fcut
            c2 = c1 + c1
            ky = _SQRT3 * rinv
            y1x = vx * ky
            y1y = vy * ky
            y1z = vz * ky
            rows = g * 16 + lanes
            gc = pl.ds(g * 16, 16)
            rshb[b][0, gc] = jnp.full((16,), 1.0, jnp.float32)
            rshb[b][1, gc] = y1x
            rshb[b][2, gc] = y1y
            rshb[b][3, gc] = y1z
            s_prev = jnp.full((16,), 0.0, jnp.float32)
            s_cur = s1
            for k in range(NB):
                rbf_k = s_cur * gsc
                kc = jnp.full((16,), k, jnp.int32)
                rbfb[b][k, gc] = rbf_k
                plsc.store_scatter(pay[b], [rows, kc], rbf_k)
                plsc.store_scatter(pay[b], [rows, kc + NB], rbf_k * y1x)
                plsc.store_scatter(pay[b], [rows, kc + 2 * NB], rbf_k * y1y)
                plsc.store_scatter(pay[b], [rows, kc + 3 * NB], rbf_k * y1z)
                s_new = c2 * s_cur - s_prev
                s_prev = s_cur
                s_cur = s_new
        pltpu.async_copy(rbfb[b], rbf_h.at[:, pl.ds(eb, C)], semw[b])
        pltpu.async_copy(rshb[b], rsh_h.at[:, pl.ds(eb, C)], semw[b])
        pltpu.sync_copy(pay[b], accs.at[dbuf[b]], add=True)

    def pair_body(up, carry):
        t0 = up * 2
        for b in (0, 1):
            t = t0 + b
            do_chunk(t, b)

            @pl.when(t + 2 < NCHUNK)
            def _prefetch():
                nb2 = base + (t + 2) * C
                pltpu.async_copy(src_h.at[pl.ds(nb2, C)], sbuf[b], semi[b])
                pltpu.async_copy(dst_h.at[pl.ds(nb2, C)], dbuf[b], semi[b])
        return carry

    lax.fori_loop(0, NCHUNK // 2, pair_body, 0)
    for b in (0, 1):
        eo = base + (NCHUNK - 2 + b) * C
        pltpu.make_async_copy(rbfb[b], rbf_h.at[:, pl.ds(eo, C)],
                              semw[b]).wait()
        pltpu.make_async_copy(rshb[b], rsh_h.at[:, pl.ds(eo, C)],
                              semw[b]).wait()
    plsc.subcore_barrier()
    obase = cid * N_PAD + stripe
    for j in range(NSTRIPE // C):
        pltpu.sync_copy(accs.at[pl.ds(stripe + j * C, C)], pay0)
        pltpu.sync_copy(pay0, acc_h.at[pl.ds(obase + j * C, C)])


_sc_kernel = functools.partial(
    pl.kernel,
    out_type=(jax.ShapeDtypeStruct((NB, E), jnp.float32),
              jax.ShapeDtypeStruct((4, E), jnp.float32),
              jax.ShapeDtypeStruct((NC * N_PAD, NS), jnp.float32)),
    mesh=_mesh,
    compiler_params=pltpu.CompilerParams(needs_layout_passes=False,
                                         use_tc_tiling_on_sc=False),
    scratch_types=[
        pltpu.VMEM((N,), jnp.float32),
        pltpu.VMEM((N,), jnp.float32),
        pltpu.VMEM((N,), jnp.float32),
        pltpu.VMEM((C,), jnp.int32),
        pltpu.VMEM((C,), jnp.int32),
        pltpu.VMEM((C,), jnp.int32),
        pltpu.VMEM((C,), jnp.int32),
        pltpu.VMEM((C, NS), jnp.float32),
        pltpu.VMEM((C, NS), jnp.float32),
        pltpu.VMEM((NB, C), jnp.float32),
        pltpu.VMEM((NB, C), jnp.float32),
        pltpu.VMEM((4, C), jnp.float32),
        pltpu.VMEM((4, C), jnp.float32),
        pltpu.VMEM_SHARED((N_PAD, NS), jnp.float32),
        pltpu.SemaphoreType.DMA,
        pltpu.SemaphoreType.DMA,
        pltpu.SemaphoreType.DMA,
        pltpu.SemaphoreType.DMA,
    ],
)(_sc_body)


NBLK = 2000
TBLK = 2560


def _xpose_body(rbf2_ref, rsh2_ref, rbf_ref, rsh_ref):
    rbf_ref[...] = rbf2_ref[...].T
    rsh_ref[...] = rsh2_ref[...].T


def _merge_body(acc0_ref, acc1_ref, atn_ref, Wa_ref, ba_ref, we_ref, be_ref,
                Wr_ref, Wp0_ref, Wp1_ref, out_ref):
    acc = acc0_ref[...] + acc1_ref[...]
    c = we_ref[0, :] + be_ref[0, :]
    scale = 1.0 / np.sqrt(NS + NV)
    hi = lax.Precision.HIGHEST
    M0 = jnp.dot(Wr_ref[:, :NS] * c[None, :], Wp0_ref[:NS, :], precision=hi) * scale
    M1 = jnp.dot(Wr_ref[:, NS:2 * NS] * c[None, :], Wp1_ref[:NS, :], precision=hi) * scale
    agg0 = jnp.dot(acc[:, :NB], M0, precision=hi)
    o1 = [jnp.dot(acc[:, NB + NB * ch:NB + NB * (ch + 1)], M1, precision=hi)
          for ch in range(3)]
    vec = jnp.stack(o1, axis=-1).reshape(acc.shape[0], NV * 3)
    atn = atn_ref[:, 0]
    onehot = (atn[:, None] == lax.broadcasted_iota(jnp.int32, (atn.shape[0], MAX_AT), 1)).astype(jnp.float32)
    scal = jnp.dot(onehot, Wa_ref[...] * (1.0 / np.sqrt(MAX_AT)),
                   precision=lax.Precision.HIGHEST) + ba_ref[0, :][None, :] + agg0
    out_ref[...] = jnp.concatenate([scal, vec], axis=-1)


def kernel(at_no, pos, edge_index, W_atom, b_atom, w_expand, b_expand, W_rbf,
           W_proj0, W_proj1):
    xs = pos[:, 1]
    ys = pos[:, 2]
    zs = pos[:, 0]
    src = edge_index[0]
    dst = edge_index[1]
    rbf_f, rsh_f, acc_f = _sc_kernel(xs, ys, zs, src, dst)
    rbf, rsh = pl.pallas_call(
        _xpose_body,
        grid=(E // TBLK,),
        in_specs=[
            pl.BlockSpec((NB, TBLK), lambda i: (0, i)),
            pl.BlockSpec((4, TBLK), lambda i: (0, i)),
        ],
        out_specs=[
            pl.BlockSpec((TBLK, NB), lambda i: (i, 0)),
            pl.BlockSpec((TBLK, 4), lambda i: (i, 0)),
        ],
        out_shape=[
            jax.ShapeDtypeStruct((E, NB), jnp.float32),
            jax.ShapeDtypeStruct((E, 4), jnp.float32),
        ],
    )(rbf_f, rsh_f)
    acc = acc_f

    full = lambda shape: pl.BlockSpec(shape, lambda i: tuple(0 for _ in shape))
    emb = pl.pallas_call(
        _merge_body,
        grid=(N // NBLK,),
        in_specs=[
            pl.BlockSpec((NBLK, NS), lambda i: (i, 0)),
            pl.BlockSpec((NBLK, NS), lambda i: (i, 0)),
            pl.BlockSpec((NBLK, 1), lambda i: (i, 0)),
            full((MAX_AT, NS)),
            full((1, NS)),
            full((1, NS)),
            full((1, NS)),
            full((NB, 2 * NS + 2 * NV)),
            full((NS + NV, NS)),
            full((NS + NV, NV)),
        ],
        out_specs=pl.BlockSpec((NBLK, NS + NV * 3), lambda i: (i, 0)),
        out_shape=jax.ShapeDtypeStruct((N, NS + NV * 3), jnp.float32),
    )(acc[:N], acc[N_PAD:N_PAD + N], at_no[:, None], W_atom, b_atom[None, :],
      w_expand[0:1, :], b_expand[None, :], W_rbf, W_proj0, W_proj1)

    return emb, rbf, rsh


# speedup vs baseline: 1.5910x; 1.5910x over previous
import functools

import jax
import jax.numpy as jnp
import numpy as np
from jax import lax
from jax.experimental import pallas as pl
from jax.experimental.pallas import tpu as pltpu
from jax.experimental.pallas import tpu_sc as plsc

N = 10000
E = 640000
MAX_AT = 119
CUTOFF = 5.0
NB = 16
NS = 64
NV = 16

NC = 2
NSUB = 16
NW = NC * NSUB
EW = E // NW
C = 80
NCHUNK = EW // C
GP = C // 16
N_PAD = 10240
NSTRIPE = N_PAD // NSUB

_SQ25 = float(np.sqrt(2.0 / CUTOFF))
_SQRT3 = float(np.sqrt(3.0))
_PI = float(np.pi)

_mesh = plsc.VectorSubcoreMesh(core_axis_name="c", subcore_axis_name="s")


def _sc_body(xs_h, ys_h, zs_h, src_h, dst_h,
             rbf_h, rsh_h, acc_h,
             xs, ys, zs,
             sbuf0, sbuf1, dbuf0, dbuf1, pay0, pay1,
             rbfb0, rbfb1, rshb0, rshb1, accs,
             semi0, semi1, semw0, semw1):
    cid = lax.axis_index("c")
    sid = lax.axis_index("s")
    wid = sid * NC + cid
    base = wid * EW
    sbuf = (sbuf0, sbuf1)
    dbuf = (dbuf0, dbuf1)
    pay = (pay0, pay1)
    rbfb = (rbfb0, rbfb1)
    rshb = (rshb0, rshb1)
    semi = (semi0, semi1)
    semw = (semw0, semw1)

    pltpu.sync_copy(xs_h, xs)
    pltpu.sync_copy(ys_h, ys)
    pltpu.sync_copy(zs_h, zs)
    stripe = sid * NSTRIPE
    zv = jnp.full((16,), 0.0, jnp.float32)
    for i in range(C):
        for j in range(NS // 16):
            pay0[i, pl.ds(j * 16, 16)] = zv
    for j in range(NSTRIPE // C):
        pltpu.sync_copy(pay0, accs.at[pl.ds(stripe + j * C, C)])
    plsc.subcore_barrier()

    lanes = lax.iota(jnp.int32, 16)

    for b in (0, 1):
        pltpu.async_copy(src_h.at[pl.ds(base + b * C, C)], sbuf[b], semi[b])
        pltpu.async_copy(dst_h.at[pl.ds(base + b * C, C)], dbuf[b], semi[b])

    def do_chunk(t, b):
        eb = base + t * C
        pltpu.make_async_copy(src_h.at[pl.ds(eb, C)], sbuf[b], semi[b]).wait()
        pltpu.make_async_copy(dst_h.at[pl.ds(eb, C)], dbuf[b], semi[b]).wait()
        @pl.when(t >= 2)
        def _drain():
            eo = eb - 2 * C
            pltpu.make_async_copy(rbfb[b], rbf_h.at[pl.ds(eo * NB, C * NB)],
                                  semw[b]).wait()
            pltpu.make_async_copy(rshb[b], rsh_h.at[pl.ds(eo * 4, C * 4)],
                                  semw[b]).wait()
        for g in range(GP):
            si = sbuf[b][pl.ds(g * 16, 16)]
            di = dbuf[b][pl.ds(g * 16, 16)]
            vx = plsc.load_gather(xs, [di]) - plsc.load_gather(xs, [si])
            vy = plsc.load_gather(ys, [di]) - plsc.load_gather(ys, [si])
            vz = plsc.load_gather(zs, [di]) - plsc.load_gather(zs, [si])
            d2 = vx * vx + vy * vy + vz * vz + 1e-12
            y = plsc.bitcast(0x5F3759DF - jnp.right_shift(plsc.bitcast(d2, jnp.int32), 1),
                             jnp.float32)
            y = y * (1.5 - 0.5 * d2 * y * y)
            y = y * (1.5 - 0.5 * d2 * y * y)
            rinv = y * (1.5 - 0.5 * d2 * y * y)
            dist = d2 * rinv
            u = dist * (1.0 / CUTOFF)
            inside = u < 1.0
            u2 = u * u
            u5 = u2 * u2 * u
            fcut = jnp.where(inside, 1.0 + u5 * (-21.0 + u * (35.0 - 15.0 * u)), 0.0)
            uc = jnp.minimum(u, 1.0)
            flip = uc > 0.5
            w = jnp.where(flip, 1.0 - uc, uc)
            r = _PI * w
            r2 = r * r
            s1 = r * (1.0 + r2 * (-1.6666654611e-1 + r2 * (8.3321608736e-3
                                                           + r2 * -1.9515295891e-4)))
            c1m = 1.0 + r2 * (-0.4999999724 + r2 * (4.166654929e-2
                                                    + r2 * (-1.388731625e-3
                                                            + r2 * 2.443315711e-5)))
            c1 = jnp.where(flip, 0.0 - c1m, c1m)
            gsc = _SQ25 * rinv * fcut
            c2 = c1 + c1
            ky = _SQRT3 * rinv
            y1x = vx * ky
            y1y = vy * ky
            y1z = vz * ky
            rows = g * 16 + lanes
            rows4 = rows * 4
            plsc.store_scatter(rshb[b], [rows4], jnp.full((16,), 1.0, jnp.float32))
            plsc.store_scatter(rshb[b], [rows4 + 1], y1x)
            plsc.store_scatter(rshb[b], [rows4 + 2], y1y)
            plsc.store_scatter(rshb[b], [rows4 + 3], y1z)
            rowsNB = rows * NB
            s_prev = jnp.full((16,), 0.0, jnp.float32)
            s_cur = s1
            for k in range(NB):
                rbf_k = s_cur * gsc
                kc = jnp.full((16,), k, jnp.int32)
                plsc.store_scatter(rbfb[b], [rowsNB + k], rbf_k)
                plsc.store_scatter(pay[b], [rows, kc], rbf_k)
                plsc.store_scatter(pay[b], [rows, kc + NB], rbf_k * y1x)
                plsc.store_scatter(pay[b], [rows, kc + 2 * NB], rbf_k * y1y)
                plsc.store_scatter(pay[b], [rows, kc + 3 * NB], rbf_k * y1z)
                s_new = c2 * s_cur - s_prev
                s_prev = s_cur
                s_cur = s_new
        pltpu.async_copy(rbfb[b], rbf_h.at[pl.ds(eb * NB, C * NB)], semw[b])
        pltpu.async_copy(rshb[b], rsh_h.at[pl.ds(eb * 4, C * 4)], semw[b])
        pltpu.sync_copy(pay[b], accs.at[dbuf[b]], add=True)

    def pair_body(up, carry):
        t0 = up * 2
        for b in (0, 1):
            t = t0 + b
            do_chunk(t, b)

            @pl.when(t + 2 < NCHUNK)
            def _prefetch():
                nb2 = base + (t + 2) * C
                pltpu.async_copy(src_h.at[pl.ds(nb2, C)], sbuf[b], semi[b])
                pltpu.async_copy(dst_h.at[pl.ds(nb2, C)], dbuf[b], semi[b])
        return carry

    lax.fori_loop(0, NCHUNK // 2, pair_body, 0)
    for b in (0, 1):
        eo = base + (NCHUNK - 2 + b) * C
        pltpu.make_async_copy(rbfb[b], rbf_h.at[pl.ds(eo * NB, C * NB)],
                              semw[b]).wait()
        pltpu.make_async_copy(rshb[b], rsh_h.at[pl.ds(eo * 4, C * 4)],
                              semw[b]).wait()
    plsc.subcore_barrier()
    obase = cid * N_PAD + stripe
    for j in range(NSTRIPE // C):
        pltpu.sync_copy(accs.at[pl.ds(stripe + j * C, C)], pay0)
        pltpu.sync_copy(pay0, acc_h.at[pl.ds(obase + j * C, C)])


_sc_kernel = functools.partial(
    pl.kernel,
    out_type=(jax.ShapeDtypeStruct((E * NB,), jnp.float32),
              jax.ShapeDtypeStruct((E * 4,), jnp.float32),
              jax.ShapeDtypeStruct((NC * N_PAD, NS), jnp.float32)),
    mesh=_mesh,
    compiler_params=pltpu.CompilerParams(needs_layout_passes=False,
                                         use_tc_tiling_on_sc=False),
    scratch_types=[
        pltpu.VMEM((N,), jnp.float32),
        pltpu.VMEM((N,), jnp.float32),
        pltpu.VMEM((N,), jnp.float32),
        pltpu.VMEM((C,), jnp.int32),
        pltpu.VMEM((C,), jnp.int32),
        pltpu.VMEM((C,), jnp.int32),
        pltpu.VMEM((C,), jnp.int32),
        pltpu.VMEM((C, NS), jnp.float32),
        pltpu.VMEM((C, NS), jnp.float32),
        pltpu.VMEM((C * NB,), jnp.float32),
        pltpu.VMEM((C * NB,), jnp.float32),
        pltpu.VMEM((C * 4,), jnp.float32),
        pltpu.VMEM((C * 4,), jnp.float32),
        pltpu.VMEM_SHARED((N_PAD, NS), jnp.float32),
        pltpu.SemaphoreType.DMA,
        pltpu.SemaphoreType.DMA,
        pltpu.SemaphoreType.DMA,
        pltpu.SemaphoreType.DMA,
    ],
)(_sc_body)


NBLK = 2000


def _merge_body(acc0_ref, acc1_ref, atn_ref, Wa_ref, ba_ref, we_ref, be_ref,
                Wr_ref, Wp0_ref, Wp1_ref, out_ref):
    acc = acc0_ref[...] + acc1_ref[...]
    c = we_ref[0, :] + be_ref[0, :]
    scale = 1.0 / np.sqrt(NS + NV)
    hi = lax.Precision.HIGHEST
    M0 = jnp.dot(Wr_ref[:, :NS] * c[None, :], Wp0_ref[:NS, :], precision=hi) * scale
    M1 = jnp.dot(Wr_ref[:, NS:2 * NS] * c[None, :], Wp1_ref[:NS, :], precision=hi) * scale
    agg0 = jnp.dot(acc[:, :NB], M0, precision=hi)
    o1 = [jnp.dot(acc[:, NB + NB * ch:NB + NB * (ch + 1)], M1, precision=hi)
          for ch in range(3)]
    vec = jnp.stack(o1, axis=-1).reshape(acc.shape[0], NV * 3)
    atn = atn_ref[:, 0]
    onehot = (atn[:, None] == lax.broadcasted_iota(jnp.int32, (atn.shape[0], MAX_AT), 1)).astype(jnp.float32)
    scal = jnp.dot(onehot, Wa_ref[...] * (1.0 / np.sqrt(MAX_AT)),
                   precision=lax.Precision.HIGHEST) + ba_ref[0, :][None, :] + agg0
    out_ref[...] = jnp.concatenate([scal, vec], axis=-1)


def kernel(at_no, pos, edge_index, W_atom, b_atom, w_expand, b_expand, W_rbf,
           W_proj0, W_proj1):
    xs = pos[:, 1]
    ys = pos[:, 2]
    zs = pos[:, 0]
    src = edge_index[0]
    dst = edge_index[1]
    rbf_f, rsh_f, acc_f = _sc_kernel(xs, ys, zs, src, dst)
    rbf = rbf_f.reshape(E, NB)
    rsh = rsh_f.reshape(E, 4)
    acc = acc_f

    full = lambda shape: pl.BlockSpec(shape, lambda i: tuple(0 for _ in shape))
    emb = pl.pallas_call(
        _merge_body,
        grid=(N // NBLK,),
        in_specs=[
            pl.BlockSpec((NBLK, NS), lambda i: (i, 0)),
            pl.BlockSpec((NBLK, NS), lambda i: (i, 0)),
            pl.BlockSpec((NBLK, 1), lambda i: (i, 0)),
            full((MAX_AT, NS)),
            full((1, NS)),
            full((1, NS)),
            full((1, NS)),
            full((NB, 2 * NS + 2 * NV)),
            full((NS + NV, NS)),
            full((NS + NV, NV)),
        ],
        out_specs=pl.BlockSpec((NBLK, NS + NV * 3), lambda i: (i, 0)),
        out_shape=jax.ShapeDtypeStruct((N, NS + NV * 3), jnp.float32),
    )(acc[:N], acc[N_PAD:N_PAD + N], at_no[:, None], W_atom, b_atom[None, :],
      w_expand[0:1, :], b_expand[None, :], W_rbf, W_proj0, W_proj1)

    return emb, rbf, rsh
